# baseline (speedup 1.0000x reference)
import functools

import numpy as np
import jax
import jax.numpy as jnp
from jax import lax
from jax.experimental import pallas as pl
from jax.experimental.pallas import tpu as pltpu, tpu_sc as plsc

_S = 512
_C = 6
_P = 8
_NW = 32
_NL = 16


def _proj_body(x_ref, W_ref, y_ref):
    x = x_ref[...]
    y_ref[...] = lax.dot_general(W_ref[...], x.astype(jnp.bfloat16),
                                 (((1,), (1,)), ((), ())),
                                 preferred_element_type=jnp.float32)


def _make_sc_segment_sum(n, ids_off=0):
    units = n // 128
    per_w = units // _NW
    rem = units % _NW
    rows_w = per_w * 128
    ch_rows = 128
    for cu in range(1, per_w + 1):
        if per_w % cu == 0 and cu * 128 <= 3328:
            ch_rows = cu * 128
    nchunks = rows_w // ch_rows
    bank = _P * _S + 1
    acc_sz = ((_NL * bank + 127) // 128) * 128
    mesh = plsc.VectorSubcoreMesh(core_axis_name="c", subcore_axis_name="s",
                                  num_cores=2, num_subcores=16)

    @functools.partial(
        pl.kernel,
        out_type=jax.ShapeDtypeStruct((_NW, _C + 1, _S), jnp.float32),
        mesh=mesh,
        scratch_types=[
            pltpu.VMEM((_P, ch_rows), jnp.float32),
            pltpu.VMEM((rows_w,), jnp.int32),
            pltpu.VMEM((_P, 128), jnp.float32),
            pltpu.VMEM((128,), jnp.int32),
            pltpu.VMEM((acc_sz,), jnp.float32),
            pltpu.VMEM((_C + 1, _S), jnp.float32),
        ],
        compiler_params=pltpu.CompilerParams(needs_layout_passes=False),
    )
    def sc(y_hbm, ids_hbm, out_hbm, y_v, ids_v, y2_v, ids2_v, acc_v, m_v):
        wid = lax.axis_index("c") * 16 + lax.axis_index("s")
        base = wid * rows_w
        pltpu.sync_copy(ids_hbm.at[pl.ds(ids_off + base, rows_w)], ids_v)

        zeros16 = jnp.zeros((16,), jnp.float32)
        ones16 = jnp.ones((16,), jnp.float32)

        def _zero(k, c):
            for u in range(8):
                acc_v[pl.ds(k * 128 + u * 16, 16)] = zeros16
            return c

        lax.fori_loop(0, acc_sz // 128, _zero, 0)

        slotbase = lax.iota(jnp.int32, 16) * bank

        for chunk in range(nchunks):
            cb = chunk * ch_rows
            pltpu.sync_copy(y_hbm.at[:, pl.ds(base + cb, ch_rows)], y_v)

            def _step(g, c):
                ids16 = ids_v[pl.ds(cb + g * 16, 16)]
                t0 = ids16 + slotbase
                vals = [y_v[ch, pl.ds(g * 16, 16)] for ch in range(_C)]
                tgts = [t0 + ch * _S for ch in range(_C + 1)]
                for ch in range(_C):
                    plsc.addupdate_scatter(acc_v, [tgts[ch]], vals[ch])
                plsc.addupdate_scatter(acc_v, [tgts[_C]], ones16)
                return c

            lax.fori_loop(0, ch_rows // 16, _step, 0)

        if rem:
            @pl.when(wid < rem)
            def _tail():
                tbase = _NW * rows_w + wid * 128
                pltpu.sync_copy(y_hbm.at[:, pl.ds(tbase, 128)], y2_v)
                pltpu.sync_copy(ids_hbm.at[pl.ds(ids_off + tbase, 128)], ids2_v)

                def _step2(g, c):
                    ids16 = ids2_v[pl.ds(g * 16, 16)]
                    t0 = ids16 + slotbase
                    vals = [y2_v[ch, pl.ds(g * 16, 16)] for ch in range(_C)]
                    tgts = [t0 + ch * _S for ch in range(_C + 1)]
                    for ch in range(_C):
                        plsc.addupdate_scatter(acc_v, [tgts[ch]], vals[ch])
                    plsc.addupdate_scatter(acc_v, [tgts[_C]], ones16)
                    return c

                lax.fori_loop(0, 8, _step2, 0)

        def _merge(s, c):
            for ch in range(_C + 1):
                o = ch * _S + s * 16
                tot = acc_v[pl.ds(o, 16)]
                for l in range(1, _NL):
                    tot = tot + acc_v[pl.ds(l * bank + o, 16)]
                m_v[ch, pl.ds(s * 16, 16)] = tot
            return c

        lax.fori_loop(0, _S // 16, _merge, 0)
        pltpu.sync_copy(m_v, out_hbm.at[wid])

    return sc


def _chunk_sizes(n):
    u = 3584
    if n % 128 != 0 or n <= 4 * u:
        return [n]
    tail = (n % u) + u
    q = (n - tail) // u
    a = q // 3
    return [a * u, a * u, (q - 2 * a) * u, tail]


def _fin_body(*refs):
    p_refs, bias_ref, out_ref = refs[:-2], refs[-2], refs[-1]
    s = p_refs[0][...].sum(axis=0)
    for p in p_refs[1:]:
        s = s + p[...].sum(axis=0)
    cnt = jnp.clip(s[_C:_C + 1, :], 1.0, None)
    out_ref[...] = jax.nn.sigmoid(s[:_C, :] / cnt + bias_ref[...])


def kernel(x, batch, W, b):
    n, d = x.shape
    chunks = _chunk_sizes(n)

    Wp = jnp.zeros((_P, d), jnp.bfloat16).at[:_C].set(W.astype(jnp.bfloat16))
    ids32 = batch.astype(jnp.int32)

    def proj_chunk(row_off, nq):
        r = 0
        for cand in range(128, 4097, 128):
            if nq % cand == 0 and row_off % cand == 0:
                r = cand
        if r == 0:
            for cand in range(8, 4097, 8):
                if nq % cand == 0 and row_off % cand == 0:
                    r = cand
        nbq = nq // r
        off = row_off // r
        return pl.pallas_call(
            _proj_body,
            grid=(nbq,),
            in_specs=[
                pl.BlockSpec((r, d), lambda i: (i + off, 0)),
                pl.BlockSpec((_P, d), lambda i: (0, 0)),
            ],
            out_specs=pl.BlockSpec((_P, r), lambda i: (0, i)),
            out_shape=jax.ShapeDtypeStruct((_P, nq), jnp.float32),
        )(x, Wp)

    parts = []
    row_off = 0
    for nq in chunks:
        y_q = proj_chunk(row_off, nq)
        sc_q = _make_sc_segment_sum(nq, ids_off=row_off)
        parts.append(sc_q(y_q, ids32))
        row_off += nq

    bias = b.reshape(_C, 1)
    out = pl.pallas_call(
        _fin_body,
        in_specs=(
            [pl.BlockSpec((_NW, _C + 1, _S), lambda: (0, 0, 0))
             for _ in chunks]
            + [pl.BlockSpec((_C, 1), lambda: (0, 0))]
        ),
        out_specs=pl.BlockSpec((_C, _S), lambda: (0, 0)),
        out_shape=jax.ShapeDtypeStruct((_C, _S), jnp.float32),
    )(*parts, bias)
    return out.T


# speedup vs baseline: 1.0297x; 1.0297x over previous
import functools

import numpy as np
import jax
import jax.numpy as jnp
from jax import lax
from jax.experimental import pallas as pl
from jax.experimental.pallas import tpu as pltpu, tpu_sc as plsc

_S = 512
_C = 6
_P = 8
_NW = 32
_NL = 16


def _proj_body(x_ref, W_ref, y_ref):
    x = x_ref[...]
    y_ref[...] = lax.dot_general(W_ref[...], x.astype(jnp.bfloat16),
                                 (((1,), (1,)), ((), ())),
                                 preferred_element_type=jnp.float32)


def _make_sc_segment_sum(n, ids_off=0):
    units = n // 128
    per_w = units // _NW
    rem = units % _NW
    rows_w = per_w * 128
    ch_rows = 128
    for cu in range(1, per_w + 1):
        if per_w % cu == 0 and cu * 128 <= 3328:
            ch_rows = cu * 128
    nchunks = rows_w // ch_rows
    bank = _P * _S + 1
    acc_sz = ((_NL * bank + 127) // 128) * 128
    mesh = plsc.VectorSubcoreMesh(core_axis_name="c", subcore_axis_name="s",
                                  num_cores=2, num_subcores=16)

    @functools.partial(
        pl.kernel,
        out_type=jax.ShapeDtypeStruct((_NW, _C + 1, _S), jnp.float32),
        mesh=mesh,
        scratch_types=[
            pltpu.VMEM((_P, ch_rows), jnp.float32),
            pltpu.VMEM((rows_w,), jnp.int32),
            pltpu.VMEM((_P, 128), jnp.float32),
            pltpu.VMEM((128,), jnp.int32),
            pltpu.VMEM((acc_sz,), jnp.float32),
            pltpu.VMEM((_C + 1, _S), jnp.float32),
        ],
        compiler_params=pltpu.CompilerParams(needs_layout_passes=False),
    )
    def sc(y_hbm, ids_hbm, out_hbm, y_v, ids_v, y2_v, ids2_v, acc_v, m_v):
        wid = lax.axis_index("c") * 16 + lax.axis_index("s")
        base = wid * rows_w
        pltpu.sync_copy(ids_hbm.at[pl.ds(ids_off + base, rows_w)], ids_v)

        zeros16 = jnp.zeros((16,), jnp.float32)
        ones16 = jnp.ones((16,), jnp.float32)

        def _zero(k, c):
            for u in range(8):
                acc_v[pl.ds(k * 128 + u * 16, 16)] = zeros16
            return c

        lax.fori_loop(0, acc_sz // 128, _zero, 0)

        slotbase = lax.iota(jnp.int32, 16) * bank

        for chunk in range(nchunks):
            cb = chunk * ch_rows
            pltpu.sync_copy(y_hbm.at[:, pl.ds(base + cb, ch_rows)], y_v)

            def _step(g, c):
                ids16 = ids_v[pl.ds(cb + g * 16, 16)]
                t0 = ids16 + slotbase
                vals = [y_v[ch, pl.ds(g * 16, 16)] for ch in range(_C)]
                tgts = [t0 + ch * _S for ch in range(_C + 1)]
                for ch in range(_C):
                    plsc.addupdate_scatter(acc_v, [tgts[ch]], vals[ch])
                plsc.addupdate_scatter(acc_v, [tgts[_C]], ones16)
                return c

            lax.fori_loop(0, ch_rows // 16, _step, 0)

        if rem:
            @pl.when(wid < rem)
            def _tail():
                tbase = _NW * rows_w + wid * 128
                pltpu.sync_copy(y_hbm.at[:, pl.ds(tbase, 128)], y2_v)
                pltpu.sync_copy(ids_hbm.at[pl.ds(ids_off + tbase, 128)], ids2_v)

                def _step2(g, c):
                    ids16 = ids2_v[pl.ds(g * 16, 16)]
                    t0 = ids16 + slotbase
                    vals = [y2_v[ch, pl.ds(g * 16, 16)] for ch in range(_C)]
                    tgts = [t0 + ch * _S for ch in range(_C + 1)]
                    for ch in range(_C):
                        plsc.addupdate_scatter(acc_v, [tgts[ch]], vals[ch])
                    plsc.addupdate_scatter(acc_v, [tgts[_C]], ones16)
                    return c

                lax.fori_loop(0, 8, _step2, 0)

        def _merge(s, c):
            for ch in range(_C + 1):
                o = ch * _S + s * 16
                tot = acc_v[pl.ds(o, 16)]
                for l in range(1, _NL):
                    tot = tot + acc_v[pl.ds(l * bank + o, 16)]
                m_v[ch, pl.ds(s * 16, 16)] = tot
            return c

        lax.fori_loop(0, _S // 16, _merge, 0)
        pltpu.sync_copy(m_v, out_hbm.at[wid])

    return sc


def _chunk_sizes(n):
    if n % 512 != 0 or n <= 16384:
        return [n]
    return [n // 4] * 4


def _fin_body(*refs):
    p_refs, bias_ref, out_ref = refs[:-2], refs[-2], refs[-1]
    s = p_refs[0][...].sum(axis=0)
    for p in p_refs[1:]:
        s = s + p[...].sum(axis=0)
    cnt = jnp.clip(s[_C:_C + 1, :], 1.0, None)
    out_ref[...] = jax.nn.sigmoid(s[:_C, :] / cnt + bias_ref[...])


def kernel(x, batch, W, b):
    n, d = x.shape
    chunks = _chunk_sizes(n)

    Wp = jnp.zeros((_P, d), jnp.bfloat16).at[:_C].set(W.astype(jnp.bfloat16))
    ids32 = batch.astype(jnp.int32)

    def proj_chunk(row_off, nq):
        r = 0
        for cand in range(128, 4097, 128):
            if nq % cand == 0 and row_off % cand == 0:
                r = cand
        if r == 0:
            for cand in range(8, 4097, 8):
                if nq % cand == 0 and row_off % cand == 0:
                    r = cand
        nbq = nq // r
        off = row_off // r
        return pl.pallas_call(
            _proj_body,
            grid=(nbq,),
            in_specs=[
                pl.BlockSpec((r, d), lambda i: (i + off, 0)),
                pl.BlockSpec((_P, d), lambda i: (0, 0)),
            ],
            out_specs=pl.BlockSpec((_P, r), lambda i: (0, i)),
            out_shape=jax.ShapeDtypeStruct((_P, nq), jnp.float32),
        )(x, Wp)

    parts = []
    row_off = 0
    for nq in chunks:
        y_q = proj_chunk(row_off, nq)
        sc_q = _make_sc_segment_sum(nq, ids_off=row_off)
        parts.append(sc_q(y_q, ids32))
        row_off += nq

    bias = b.reshape(_C, 1)
    out = pl.pallas_call(
        _fin_body,
        in_specs=(
            [pl.BlockSpec((_NW, _C + 1, _S), lambda: (0, 0, 0))
             for _ in chunks]
            + [pl.BlockSpec((_C, 1), lambda: (0, 0))]
        ),
        out_specs=pl.BlockSpec((_C, _S), lambda: (0, 0)),
        out_shape=jax.ShapeDtypeStruct((_C, _S), jnp.float32),
    )(*parts, bias)
    return out.T
